# Initial kernel scaffold; baseline (speedup 1.0000x reference)
import jax
import jax.numpy as jnp
from jax.experimental import pallas as pl


def kernel(z, codebook):
    raise NotImplementedError("write your pallas kernel here")


# speedup vs baseline: 1.2162x; 1.2162x over previous
import functools

import jax
import jax.numpy as jnp
from jax.experimental import pallas as pl
from jax.experimental.pallas import tpu as pltpu

_NUM_CODES = 512
_CODE_DIM = 32
_COMMITMENT_COST = 0.25
_ROWS_PER_BLOCK = 2


def _vq_block_kernel(z_ref, cb_ref, zq_ref, idx_ref, loss_ref):
    i = pl.program_id(0)
    bt = z_ref.shape[0] * z_ref.shape[1]
    z = z_ref[...].reshape(bt, _CODE_DIM)
    cb = cb_ref[...]

    z2 = jnp.sum(z * z, axis=1, keepdims=True)
    c2 = jnp.sum(cb * cb, axis=1)
    dots = jax.lax.dot_general(
        z, cb, (((1,), (1,)), ((), ())),
        preferred_element_type=jnp.float32)
    dist = z2 + c2[None, :] - 2.0 * dots
    m = jnp.min(dist, axis=1, keepdims=True)
    iota = jax.lax.broadcasted_iota(jnp.int32, dist.shape, 1)
    idx = jnp.min(jnp.where(dist == m, iota, _NUM_CODES), axis=1).astype(jnp.int32)

    onehot = (idx[:, None] == jax.lax.broadcasted_iota(
        jnp.int32, (1, _NUM_CODES), 1)).astype(jnp.float32)
    zq = jax.lax.dot_general(
        onehot, cb, (((1,), (0,)), ((), ())),
        preferred_element_type=jnp.float32,
        precision=jax.lax.Precision.HIGHEST)

    zq_ref[...] = zq.reshape(zq_ref.shape)
    idx_ref[...] = idx.reshape(idx_ref.shape)

    r = z - zq
    part = jnp.sum(r * r)

    @pl.when(i == 0)
    def _init():
        loss_ref[0, 0] = part

    @pl.when(i != 0)
    def _acc():
        loss_ref[0, 0] = loss_ref[0, 0] + part


@functools.partial(jax.jit, static_argnames=())
def kernel(z, codebook):
    n_rows, row_len, d = z.shape
    grid = n_rows // _ROWS_PER_BLOCK
    zq, idx, loss_sum = pl.pallas_call(
        _vq_block_kernel,
        grid=(grid,),
        in_specs=[
            pl.BlockSpec((_ROWS_PER_BLOCK, row_len, d), lambda i: (i, 0, 0)),
            pl.BlockSpec((_NUM_CODES, _CODE_DIM), lambda i: (0, 0)),
        ],
        out_specs=[
            pl.BlockSpec((_ROWS_PER_BLOCK, row_len, d), lambda i: (i, 0, 0)),
            pl.BlockSpec((_ROWS_PER_BLOCK, 1, row_len), lambda i: (i, 0, 0)),
            pl.BlockSpec(memory_space=pltpu.SMEM),
        ],
        out_shape=[
            jax.ShapeDtypeStruct(z.shape, jnp.float32),
            jax.ShapeDtypeStruct((n_rows, 1, row_len), jnp.int32),
            jax.ShapeDtypeStruct((1, 1), jnp.float32),
        ],
    )(z, codebook)
    loss = (_COMMITMENT_COST / (n_rows * row_len * d)) * loss_sum[0, 0]
    return (zq, loss, idx.reshape(n_rows, row_len))


# speedup vs baseline: 2.2433x; 1.8445x over previous
import functools

import jax
import jax.numpy as jnp
from jax.experimental import pallas as pl
from jax.experimental.pallas import tpu as pltpu

_NUM_CODES = 512
_CODE_DIM = 32
_COMMITMENT_COST = 0.25
_ROWS_PER_BLOCK = 2


def _vq_block_kernel(z_ref, cb_ref, zq_ref, idx_ref, loss_ref):
    i = pl.program_id(0)
    bt = z_ref.shape[0] * z_ref.shape[1]
    z = z_ref[...].reshape(bt, _CODE_DIM)
    cb = cb_ref[...]

    z2 = jnp.sum(z * z, axis=1, keepdims=True)
    c2 = jnp.sum(cb * cb, axis=1)
    dots = jax.lax.dot_general(
        z, cb, (((1,), (1,)), ((), ())),
        preferred_element_type=jnp.float32)
    dist = z2 + c2[None, :] - 2.0 * dots
    m = jnp.min(dist, axis=1, keepdims=True)
    iota_f = jax.lax.broadcasted_iota(jnp.int32, dist.shape, 1).astype(jnp.float32)
    idx_f = jnp.min(jnp.where(dist == m, iota_f, float(_NUM_CODES)), axis=1)
    idx = idx_f.astype(jnp.int32)

    onehot = (idx_f[:, None] == jax.lax.broadcasted_iota(
        jnp.int32, (1, _NUM_CODES), 1).astype(jnp.float32)).astype(jnp.float32)
    zq = jax.lax.dot_general(
        onehot, cb, (((1,), (0,)), ((), ())),
        preferred_element_type=jnp.float32)

    zq_ref[...] = zq.reshape(zq_ref.shape)
    idx_ref[...] = idx.reshape(idx_ref.shape)

    part = jnp.sum(m)

    @pl.when(i == 0)
    def _init():
        loss_ref[0, 0] = part

    @pl.when(i != 0)
    def _acc():
        loss_ref[0, 0] = loss_ref[0, 0] + part


@functools.partial(jax.jit, static_argnames=())
def kernel(z, codebook):
    n_rows, row_len, d = z.shape
    grid = n_rows // _ROWS_PER_BLOCK
    zq, idx, loss_sum = pl.pallas_call(
        _vq_block_kernel,
        grid=(grid,),
        in_specs=[
            pl.BlockSpec((_ROWS_PER_BLOCK, row_len, d), lambda i: (i, 0, 0)),
            pl.BlockSpec((_NUM_CODES, _CODE_DIM), lambda i: (0, 0)),
        ],
        out_specs=[
            pl.BlockSpec((_ROWS_PER_BLOCK, row_len, d), lambda i: (i, 0, 0)),
            pl.BlockSpec((_ROWS_PER_BLOCK, 1, row_len), lambda i: (i, 0, 0)),
            pl.BlockSpec(memory_space=pltpu.SMEM),
        ],
        out_shape=[
            jax.ShapeDtypeStruct(z.shape, jnp.float32),
            jax.ShapeDtypeStruct((n_rows, 1, row_len), jnp.int32),
            jax.ShapeDtypeStruct((1, 1), jnp.float32),
        ],
    )(z, codebook)
    loss = (_COMMITMENT_COST / (n_rows * row_len * d)) * loss_sum[0, 0]
    return (zq, loss, idx.reshape(n_rows, row_len))


# speedup vs baseline: 2.5304x; 1.1280x over previous
import functools

import jax
import jax.numpy as jnp
from jax.experimental import pallas as pl
from jax.experimental.pallas import tpu as pltpu

_NUM_CODES = 512
_CODE_DIM = 32
_COMMITMENT_COST = 0.25
_ROWS_PER_BLOCK = 2


def _vq_block_kernel(z_ref, z2_ref, cb_ref, zq_ref, idx_ref, loss_ref):
    i = pl.program_id(0)
    bt = z_ref.shape[0] * z_ref.shape[1]
    z = z_ref[...].reshape(bt, _CODE_DIM)
    z2 = z2_ref[...]
    cb = cb_ref[...]

    c2 = jnp.sum(cb * cb, axis=1, keepdims=True)
    dots_t = jax.lax.dot_general(
        cb, z, (((1,), (1,)), ((), ())),
        preferred_element_type=jnp.float32)
    dist_t = (z2 + c2) - 2.0 * dots_t

    m = jnp.min(dist_t, axis=0, keepdims=True)
    iota_f = jax.lax.broadcasted_iota(
        jnp.int32, dist_t.shape, 0).astype(jnp.float32)
    idx_f = jnp.min(jnp.where(dist_t == m, iota_f,
                              float(_NUM_CODES)), axis=0)
    idx = idx_f.astype(jnp.int32)

    onehot_t = (iota_f == idx_f[None, :]).astype(jnp.float32)
    zq = jax.lax.dot_general(
        onehot_t, cb, (((0,), (0,)), ((), ())),
        preferred_element_type=jnp.float32)

    zq_ref[...] = zq.reshape(zq_ref.shape)
    idx_ref[...] = idx.reshape(idx_ref.shape)

    part = jnp.sum(m)

    @pl.when(i == 0)
    def _init():
        loss_ref[0, 0] = part

    @pl.when(i != 0)
    def _acc():
        loss_ref[0, 0] = loss_ref[0, 0] + part


@functools.partial(jax.jit, static_argnames=())
def kernel(z, codebook):
    n_rows, row_len, d = z.shape
    n_tok = n_rows * row_len
    bt = _ROWS_PER_BLOCK * row_len
    grid = n_rows // _ROWS_PER_BLOCK
    z2 = jnp.sum(z * z, axis=-1).reshape(1, n_tok)
    zq, idx, loss_sum = pl.pallas_call(
        _vq_block_kernel,
        grid=(grid,),
        in_specs=[
            pl.BlockSpec((_ROWS_PER_BLOCK, row_len, d), lambda i: (i, 0, 0)),
            pl.BlockSpec((1, bt), lambda i: (0, i)),
            pl.BlockSpec((_NUM_CODES, _CODE_DIM), lambda i: (0, 0)),
        ],
        out_specs=[
            pl.BlockSpec((_ROWS_PER_BLOCK, row_len, d), lambda i: (i, 0, 0)),
            pl.BlockSpec((_ROWS_PER_BLOCK, 1, row_len), lambda i: (i, 0, 0)),
            pl.BlockSpec(memory_space=pltpu.SMEM),
        ],
        out_shape=[
            jax.ShapeDtypeStruct(z.shape, jnp.float32),
            jax.ShapeDtypeStruct((n_rows, 1, row_len), jnp.int32),
            jax.ShapeDtypeStruct((1, 1), jnp.float32),
        ],
    )(z, z2, codebook)
    loss = (_COMMITMENT_COST / (n_tok * d)) * loss_sum[0, 0]
    return (zq, loss, idx.reshape(n_rows, row_len))


# speedup vs baseline: 2.6025x; 1.0285x over previous
import functools

import jax
import jax.numpy as jnp
from jax.experimental import pallas as pl
from jax.experimental.pallas import tpu as pltpu

_NUM_CODES = 512
_CODE_DIM = 32
_COMMITMENT_COST = 0.25
_ROWS_PER_BLOCK = 8


def _vq_block_kernel(z_ref, z2_ref, cb_ref, zq_ref, idx_ref, loss_ref):
    i = pl.program_id(0)
    bt = z_ref.shape[0] * z_ref.shape[1]
    z = z_ref[...].reshape(bt, _CODE_DIM)
    z2 = z2_ref[...]
    cb = cb_ref[...]

    c2 = jnp.sum(cb * cb, axis=1, keepdims=True)
    dots_t = jax.lax.dot_general(
        cb, z, (((1,), (1,)), ((), ())),
        preferred_element_type=jnp.float32)
    dist_t = (z2 + c2) - 2.0 * dots_t

    m = jnp.min(dist_t, axis=0, keepdims=True)
    iota_f = jax.lax.broadcasted_iota(
        jnp.int32, dist_t.shape, 0).astype(jnp.float32)
    idx_f = jnp.min(jnp.where(dist_t == m, iota_f,
                              float(_NUM_CODES)), axis=0)
    idx = idx_f.astype(jnp.int32)

    onehot_t = (iota_f == idx_f[None, :]).astype(jnp.float32)
    zq = jax.lax.dot_general(
        onehot_t, cb, (((0,), (0,)), ((), ())),
        preferred_element_type=jnp.float32)

    zq_ref[...] = zq.reshape(zq_ref.shape)
    idx_ref[...] = idx.reshape(idx_ref.shape)

    part = jnp.sum(m)

    @pl.when(i == 0)
    def _init():
        loss_ref[0, 0] = part

    @pl.when(i != 0)
    def _acc():
        loss_ref[0, 0] = loss_ref[0, 0] + part


@functools.partial(jax.jit, static_argnames=())
def kernel(z, codebook):
    n_rows, row_len, d = z.shape
    n_tok = n_rows * row_len
    bt = _ROWS_PER_BLOCK * row_len
    grid = n_rows // _ROWS_PER_BLOCK
    z2 = jnp.sum(z * z, axis=-1).reshape(1, n_tok)
    zq, idx, loss_sum = pl.pallas_call(
        _vq_block_kernel,
        grid=(grid,),
        in_specs=[
            pl.BlockSpec((_ROWS_PER_BLOCK, row_len, d), lambda i: (i, 0, 0)),
            pl.BlockSpec((1, bt), lambda i: (0, i)),
            pl.BlockSpec((_NUM_CODES, _CODE_DIM), lambda i: (0, 0)),
        ],
        out_specs=[
            pl.BlockSpec((_ROWS_PER_BLOCK, row_len, d), lambda i: (i, 0, 0)),
            pl.BlockSpec((_ROWS_PER_BLOCK, 1, row_len), lambda i: (i, 0, 0)),
            pl.BlockSpec(memory_space=pltpu.SMEM),
        ],
        out_shape=[
            jax.ShapeDtypeStruct(z.shape, jnp.float32),
            jax.ShapeDtypeStruct((n_rows, 1, row_len), jnp.int32),
            jax.ShapeDtypeStruct((1, 1), jnp.float32),
        ],
    )(z, z2, codebook)
    loss = (_COMMITMENT_COST / (n_tok * d)) * loss_sum[0, 0]
    return (zq, loss, idx.reshape(n_rows, row_len))


# speedup vs baseline: 2.6046x; 1.0008x over previous
import functools

import jax
import jax.numpy as jnp
from jax.experimental import pallas as pl
from jax.experimental.pallas import tpu as pltpu

_NUM_CODES = 512
_CODE_DIM = 32
_COMMITMENT_COST = 0.25
_ROWS_PER_BLOCK = 8


def _vq_block_kernel(z_ref, cb_ref, zq_ref, idx_ref, loss_ref):
    i = pl.program_id(0)
    bt = z_ref.shape[0] * z_ref.shape[1]
    z = z_ref[...].reshape(bt, _CODE_DIM)
    cb = cb_ref[...]

    r32 = jax.lax.broadcasted_iota(jnp.int32, (_CODE_DIM, _CODE_DIM), 0)
    c32 = jax.lax.broadcasted_iota(jnp.int32, (_CODE_DIM, _CODE_DIM), 1)
    eye = (r32 == c32).astype(jnp.float32)
    zsq_t = jax.lax.dot_general(
        eye, z * z, (((1,), (1,)), ((), ())),
        preferred_element_type=jnp.float32)
    acc = zsq_t
    w = _CODE_DIM
    while w > 1:
        w //= 2
        acc = acc[:w, :] + acc[w:2 * w, :]
    z2 = acc

    c2 = jnp.sum(cb * cb, axis=1, keepdims=True)
    dots_t = jax.lax.dot_general(
        cb, z, (((1,), (1,)), ((), ())),
        preferred_element_type=jnp.float32)
    dist_t = (z2 + c2) - 2.0 * dots_t

    m = jnp.min(dist_t, axis=0, keepdims=True)
    iota_f = jax.lax.broadcasted_iota(
        jnp.int32, dist_t.shape, 0).astype(jnp.float32)
    idx_f = jnp.min(jnp.where(dist_t == m, iota_f,
                              float(_NUM_CODES)), axis=0)
    idx = idx_f.astype(jnp.int32)

    onehot_t = (iota_f == idx_f[None, :]).astype(jnp.float32)
    zq = jax.lax.dot_general(
        onehot_t, cb, (((0,), (0,)), ((), ())),
        preferred_element_type=jnp.float32)

    zq_ref[...] = zq.reshape(zq_ref.shape)
    idx_ref[...] = idx.reshape(idx_ref.shape)

    part = jnp.sum(m)

    @pl.when(i == 0)
    def _init():
        loss_ref[0, 0] = part

    @pl.when(i != 0)
    def _acc():
        loss_ref[0, 0] = loss_ref[0, 0] + part


@functools.partial(jax.jit, static_argnames=())
def kernel(z, codebook):
    n_rows, row_len, d = z.shape
    n_tok = n_rows * row_len
    bt = _ROWS_PER_BLOCK * row_len
    grid = n_rows // _ROWS_PER_BLOCK
    zq, idx, loss_sum = pl.pallas_call(
        _vq_block_kernel,
        grid=(grid,),
        in_specs=[
            pl.BlockSpec((_ROWS_PER_BLOCK, row_len, d), lambda i: (i, 0, 0)),
            pl.BlockSpec((_NUM_CODES, _CODE_DIM), lambda i: (0, 0)),
        ],
        out_specs=[
            pl.BlockSpec((_ROWS_PER_BLOCK, row_len, d), lambda i: (i, 0, 0)),
            pl.BlockSpec((_ROWS_PER_BLOCK, 1, row_len), lambda i: (i, 0, 0)),
            pl.BlockSpec(memory_space=pltpu.SMEM),
        ],
        out_shape=[
            jax.ShapeDtypeStruct(z.shape, jnp.float32),
            jax.ShapeDtypeStruct((n_rows, 1, row_len), jnp.int32),
            jax.ShapeDtypeStruct((1, 1), jnp.float32),
        ],
    )(z, codebook)
    loss = (_COMMITMENT_COST / (n_tok * d)) * loss_sum[0, 0]
    return (zq, loss, idx.reshape(n_rows, row_len))
